# baseline (speedup 1.0000x reference)
import functools

import jax
import jax.numpy as jnp
from jax import lax
from jax.experimental import pallas as pl
from jax.experimental.pallas import tpu as pltpu
from jax.experimental.pallas import tpu_sc as plsc

D = 64
SCALE = 8.0
NC = 2
NS = 16
NW = NC * NS
ROWL = 200
SPLIT = 96
LENS = (SPLIT, ROWL - SPLIT)
OFFS = (0, SPLIT)


def _make_emb(NR: int):
    rows_per_tile = NR // NW
    nslots = 2 * rows_per_tile
    mesh = plsc.VectorSubcoreMesh(core_axis_name="c", subcore_axis_name="s")

    @functools.partial(
        pl.kernel,
        mesh=mesh,
        out_type=jax.ShapeDtypeStruct((NR, ROWL, D), jnp.float32),
        scratch_types=[
            pltpu.VMEM((rows_per_tile, ROWL), jnp.int32),
            pltpu.VMEM((2, 128), jnp.int32),
            pltpu.VMEM((2, LENS[1], D), jnp.float32),
            pltpu.VMEM((2, LENS[1], D), jnp.float32),
            pltpu.SemaphoreType.DMA((2,)),
            pltpu.SemaphoreType.DMA((2,)),
        ],
        compiler_params=pltpu.CompilerParams(use_tc_tiling_on_sc=False),
    )
    def emb(x_hbm, tab_hbm, out_hbm, idx_v, list_v, raw_v, cmp_v, gsem, ssem):
        wid = lax.axis_index("s") * NC + lax.axis_index("c")
        row0 = wid * rows_per_tile
        pltpu.sync_copy(x_hbm.at[pl.ds(row0, rows_per_tile)], idx_v)

        def start_gather(g, b):
            xr = lax.div(g, 2)
            off, ln = OFFS[b], LENS[b]
            ngrp = (ln + 15) // 16
            for m in range(ngrp):
                c0 = min(m * 16, ln - 16)
                list_v[b, pl.ds(c0, 16)] = idx_v[xr, pl.ds(off + c0, 16)]
            pltpu.async_copy(
                tab_hbm.at[list_v.at[b].at[pl.ds(0, ln)]],
                raw_v.at[b].at[pl.ds(0, ln)],
                gsem.at[b],
            )

        def wait_gather(b):
            pltpu.make_async_copy(
                tab_hbm.at[pl.ds(0, LENS[b])],
                raw_v.at[b].at[pl.ds(0, LENS[b])],
                gsem.at[b],
            ).wait()

        def scale(b):
            ln = LENS[b]

            def grp(m, c):
                r0 = m * 4
                for rr in range(4):
                    for k in range(D // 16):
                        sl = (r0 + rr, pl.ds(k * 16, 16))
                        cmp_v[(b,) + sl] = raw_v[(b,) + sl] * SCALE
                return c

            lax.fori_loop(0, ln // 4, grp, 0)

        def start_scatter(g, b):
            xr = lax.div(g, 2)
            pltpu.async_copy(
                cmp_v.at[b].at[pl.ds(0, LENS[b])],
                out_hbm.at[row0 + xr].at[pl.ds(OFFS[b], LENS[b])],
                ssem.at[b],
            )

        def wait_scatter(b):
            pltpu.make_async_copy(
                out_hbm.at[0].at[pl.ds(0, LENS[b])],
                cmp_v.at[b].at[pl.ds(0, LENS[b])],
                ssem.at[b],
            ).wait()

        start_gather(0, 0)

        def slot(g, carry):
            def per_type(b):
                @pl.when(lax.rem(g, 2) == b)
                def _():
                    @pl.when(g + 1 < nslots)
                    def _():
                        start_gather(g + 1, 1 - b)

                    wait_gather(b)

                    @pl.when(g >= 2)
                    def _():
                        wait_scatter(b)

                    scale(b)
                    start_scatter(g, b)

            per_type(0)
            per_type(1)
            return carry

        lax.fori_loop(0, nslots, slot, 0)

        wait_scatter(0)
        wait_scatter(1)

    return emb


def kernel(x, table):
    return _make_emb(x.shape[0])(x.astype(jnp.int32), table)


# speedup vs baseline: 1.0591x; 1.0591x over previous
import functools

import jax
import jax.numpy as jnp
from jax import lax
from jax.experimental import pallas as pl
from jax.experimental.pallas import tpu as pltpu
from jax.experimental.pallas import tpu_sc as plsc

D = 64
SCALE = 8.0
NC = 2
NS = 16
NW = NC * NS
ROWL = 200
SPLIT = 96
LENS = (SPLIT, ROWL - SPLIT)
OFFS = (0, SPLIT)

_GDN = lax.GatherDimensionNumbers(
    offset_dims=(), collapsed_slice_dims=(0,), start_index_map=(0,)
)


def _splat_lane(vec16, lane):
    idx = jnp.full((16, 1), lane, jnp.int32)
    return lax.gather(
        vec16, idx, _GDN, (1,), mode=lax.GatherScatterMode.PROMISE_IN_BOUNDS
    )


def _make_emb(NR: int):
    rows_per_tile = NR // NW
    nslots = 2 * rows_per_tile
    mesh = plsc.VectorSubcoreMesh(core_axis_name="c", subcore_axis_name="s")

    @functools.partial(
        pl.kernel,
        mesh=mesh,
        out_type=jax.ShapeDtypeStruct((NR, ROWL, D), jnp.float32),
        scratch_types=[
            pltpu.VMEM((rows_per_tile, ROWL), jnp.int32),
            pltpu.VMEM((2, 128), jnp.int32),
            pltpu.VMEM((2, LENS[1], 2 * D), jnp.float32),
            pltpu.VMEM((2, LENS[1], D), jnp.float32),
            pltpu.SemaphoreType.DMA((2,)),
            pltpu.SemaphoreType.DMA((2,)),
        ],
        compiler_params=pltpu.CompilerParams(
            use_tc_tiling_on_sc=True,
            needs_layout_passes=False,
            skip_device_barrier=True,
            disable_bounds_checks=True,
            disable_semaphore_checks=True,
        ),
    )
    def emb(x_hbm, tab_hbm, out_hbm, idx_v, list_v, raw_v, cmp_v, gsem, ssem):
        wid = lax.axis_index("s") * NC + lax.axis_index("c")
        row0 = wid * rows_per_tile
        pltpu.sync_copy(x_hbm.at[pl.ds(row0, rows_per_tile)], idx_v)

        def start_gather(g, b):
            xr = lax.div(g, 2)
            off, ln = OFFS[b], LENS[b]
            ngrp = (ln + 15) // 16
            for m in range(ngrp):
                c0 = min(m * 16, ln - 16)
                list_v[b, pl.ds(c0, 16)] = lax.shift_right_logical(
                    idx_v[xr, pl.ds(off + c0, 16)], 1
                )
            pltpu.async_copy(
                tab_hbm.at[list_v.at[b].at[pl.ds(0, ln)]],
                raw_v.at[b].at[pl.ds(0, ln)],
                gsem.at[b],
            )

        def wait_gather(b):
            pltpu.make_async_copy(
                tab_hbm.at[pl.ds(0, LENS[b])],
                raw_v.at[b].at[pl.ds(0, LENS[b])],
                gsem.at[b],
            ).wait()

        def select_scale(g, b):
            xr = lax.div(g, 2)
            off, ln = OFFS[b], LENS[b]
            ngrp = (ln + 15) // 16

            def grp(m, c):
                r0 = jnp.minimum(m * 16, ln - 16)
                par16 = idx_v[xr, pl.ds(off + r0, 16)] & jnp.int32(1)
                for lane in range(16):
                    r = r0 + lane
                    sel = _splat_lane(par16, lane) == jnp.int32(1)
                    for k in range(D // 16):
                        lo = raw_v[b, r, pl.ds(k * 16, 16)]
                        hi = raw_v[b, r, pl.ds(D + k * 16, 16)]
                        cmp_v[b, r, pl.ds(k * 16, 16)] = (
                            jnp.where(sel, hi, lo) * SCALE
                        )
                return c

            lax.fori_loop(0, ngrp, grp, 0)

        def start_scatter(g, b):
            xr = lax.div(g, 2)
            pltpu.async_copy(
                cmp_v.at[b].at[pl.ds(0, LENS[b])],
                out_hbm.at[row0 + xr].at[pl.ds(OFFS[b], LENS[b])],
                ssem.at[b],
            )

        def wait_scatter(b):
            pltpu.make_async_copy(
                out_hbm.at[0].at[pl.ds(0, LENS[b])],
                cmp_v.at[b].at[pl.ds(0, LENS[b])],
                ssem.at[b],
            ).wait()

        start_gather(0, 0)

        def slot(g, carry):
            def per_type(b):
                @pl.when(lax.rem(g, 2) == b)
                def _():
                    @pl.when(g + 1 < nslots)
                    def _():
                        start_gather(g + 1, 1 - b)

                    wait_gather(b)

                    @pl.when(g >= 2)
                    def _():
                        wait_scatter(b)

                    select_scale(g, b)
                    start_scatter(g, b)

            per_type(0)
            per_type(1)
            return carry

        lax.fori_loop(0, nslots, slot, 0)

        wait_scatter(0)
        wait_scatter(1)

    return emb


def kernel(x, table):
    tab2 = table.reshape(table.shape[0] // 2, 2 * D)
    return _make_emb(x.shape[0])(x.astype(jnp.int32), tab2)
